# baseline (speedup 1.0000x reference)
import functools

import jax
import jax.numpy as jnp
from jax.experimental import pallas as pl
from jax.experimental.pallas import tpu as pltpu


def _pick_bb(batch):
    return 512 if batch % 512 == 0 else batch


def _pick_bc(c_pad):
    for bc in (384, 256, 128):
        if c_pad % bc == 0:
            return bc
    return c_pad


def _main_body(nc, num_classes, cls_ref, wb_ref, rp_ref, rq_ref,
               phi_ref, plo_ref, qhi_ref, qlo_ref, loss_ref, acc_ref,
               m_ref, a_ref, s_ref, t_ref):
    i = pl.program_id(0)
    j = pl.program_id(1)
    w = wb_ref[0]
    b = wb_ref[1]
    bc = qhi_ref.shape[0]

    nt = (((1,), (1,)), ((), ()))
    S = jax.lax.dot_general(phi_ref[...], qhi_ref[...], nt,
                            preferred_element_type=jnp.float32)
    S += jax.lax.dot_general(phi_ref[...], qlo_ref[...], nt,
                             preferred_element_type=jnp.float32)
    S += jax.lax.dot_general(plo_ref[...], qhi_ref[...], nt,
                             preferred_element_type=jnp.float32)

    rp = rp_ref[...]
    rq = rq_ref[0, :]
    l = (S * rp[:, None]) * (rq * w)[None, :] + b
    cglob = j * bc + jax.lax.broadcasted_iota(jnp.int32, (1, bc), 1)
    l = jnp.where(cglob < num_classes, l, -1e30)

    K = jnp.abs(w) + jnp.abs(b)
    s_tile = jnp.sum(jnp.exp(l - K), axis=1)
    m_tile = jnp.max(l, axis=1)
    idx_tile = jnp.min(jnp.where(l == m_tile[:, None], cglob, jnp.int32(2**30)),
                       axis=1)
    clsv = cls_ref[...]
    t_tile = jnp.sum(jnp.where(cglob == clsv[:, None], l, 0.0), axis=1)

    @pl.when(j == 0)
    def _():
        m_ref[...] = m_tile
        a_ref[...] = idx_tile
        s_ref[...] = s_tile
        t_ref[...] = t_tile

    @pl.when(j > 0)
    def _():
        m_old = m_ref[...]
        upd = m_tile > m_old
        m_ref[...] = jnp.maximum(m_old, m_tile)
        a_ref[...] = jnp.where(upd, idx_tile, a_ref[...])
        s_ref[...] = s_ref[...] + s_tile
        t_ref[...] = t_ref[...] + t_tile

    @pl.when(j == nc - 1)
    def _():
        row_loss = jnp.log(s_ref[...]) + K - t_ref[...]
        part_loss = jnp.sum(row_loss)
        part_acc = jnp.sum((a_ref[...] == clsv).astype(jnp.float32))

        @pl.when(i == 0)
        def _():
            loss_ref[0, 0] = part_loss
            acc_ref[0, 0] = part_acc

        @pl.when(i > 0)
        def _():
            loss_ref[0, 0] = loss_ref[0, 0] + part_loss
            acc_ref[0, 0] = acc_ref[0, 0] + part_acc


def _fused_loss(cls, wb, rp, rq, phi, plo, qhi, qlo, num_classes,
                interpret=False):
    batch, dim = phi.shape
    c_pad = qhi.shape[0]
    bb = _pick_bb(batch)
    bc = _pick_bc(c_pad)
    nb = batch // bb
    nc = c_pad // bc
    body = functools.partial(_main_body, nc, num_classes)
    loss_sum, acc_sum = pl.pallas_call(
        body,
        grid=(nb, nc),
        in_specs=[
            pl.BlockSpec((bb,), lambda i, j: (i,)),
            pl.BlockSpec(memory_space=pltpu.SMEM),
            pl.BlockSpec((bb,), lambda i, j: (i,)),
            pl.BlockSpec((1, bc), lambda i, j: (0, j)),
            pl.BlockSpec((bb, dim), lambda i, j: (i, 0)),
            pl.BlockSpec((bb, dim), lambda i, j: (i, 0)),
            pl.BlockSpec((bc, dim), lambda i, j: (j, 0)),
            pl.BlockSpec((bc, dim), lambda i, j: (j, 0)),
        ],
        out_specs=[
            pl.BlockSpec(memory_space=pltpu.SMEM),
            pl.BlockSpec(memory_space=pltpu.SMEM),
        ],
        out_shape=[
            jax.ShapeDtypeStruct((1, 1), jnp.float32),
            jax.ShapeDtypeStruct((1, 1), jnp.float32),
        ],
        scratch_shapes=[
            pltpu.VMEM((bb,), jnp.float32),
            pltpu.VMEM((bb,), jnp.int32),
            pltpu.VMEM((bb,), jnp.float32),
            pltpu.VMEM((bb,), jnp.float32),
        ],
        compiler_params=pltpu.CompilerParams(
            dimension_semantics=("arbitrary", "arbitrary"),
        ),
        interpret=interpret,
    )(cls, wb, rp, rq.reshape(1, c_pad), phi, plo, qhi, qlo)
    nloss = loss_sum[0, 0] / batch
    prec1 = acc_sum[0, 0] / batch * 100.0
    return nloss, prec1


def kernel(x, epoch, classes, w, b, queue, queue_ptr):
    batch = x.shape[0]
    num_classes = queue.shape[0]
    cls = classes[0]
    iota = jnp.arange(batch, dtype=jnp.int32)
    winner = jnp.full((num_classes,), -1, jnp.int32).at[cls].max(iota)
    covered = winner >= 0
    anchors = x[:, 1, :]
    q_eff = jnp.where(covered[:, None], anchors[jnp.maximum(winner, 0)],
                      queue[:, 0, :])
    c_pad = ((num_classes + 383) // 384) * 384
    q_eff = jnp.pad(q_eff, ((0, c_pad - num_classes), (0, 0)))

    pos = x[:, 0, :]
    rp = 1.0 / jnp.maximum(jnp.linalg.norm(pos, axis=1), 1e-8)
    rq = 1.0 / jnp.maximum(jnp.linalg.norm(q_eff, axis=1), 1e-8)
    phi = pos.astype(jnp.bfloat16)
    plo = (pos - phi.astype(jnp.float32)).astype(jnp.bfloat16)
    qhi = q_eff.astype(jnp.bfloat16)
    qlo = (q_eff - qhi.astype(jnp.float32)).astype(jnp.bfloat16)

    wb = jnp.stack([w.astype(jnp.float32), b.astype(jnp.float32)])
    return _fused_loss(cls, wb, rp, rq, phi, plo, qhi, qlo, num_classes)


# speedup vs baseline: 1.1135x; 1.1135x over previous
import functools

import jax
import jax.numpy as jnp
from jax.experimental import pallas as pl
from jax.experimental.pallas import tpu as pltpu


def _pick_bb(batch):
    for bb in (1024, 512, 256, 128):
        if batch % bb == 0:
            return bb
    return batch


def _pick_bc(c_pad):
    for bc in (384, 256, 128):
        if c_pad % bc == 0:
            return bc
    return c_pad


def _main_body(nc, num_classes, cls_ref, wb_ref, rp_ref, rq_ref,
               phit_ref, plot_ref, qhi_ref, qlo_ref, loss_ref, acc_ref,
               m_ref, a_ref, s_ref, t_ref):
    i = pl.program_id(0)
    j = pl.program_id(1)
    w = wb_ref[0]
    b = wb_ref[1]
    bc = qhi_ref.shape[0]

    nn = (((1,), (0,)), ((), ()))
    S = jax.lax.dot_general(qhi_ref[...], phit_ref[...], nn,
                            preferred_element_type=jnp.float32)
    S += jax.lax.dot_general(qlo_ref[...], phit_ref[...], nn,
                             preferred_element_type=jnp.float32)
    S += jax.lax.dot_general(qhi_ref[...], plot_ref[...], nn,
                             preferred_element_type=jnp.float32)

    rp = rp_ref[...]
    rq = rq_ref[0, :]
    l = (S * rq[:, None]) * (rp * w)[None, :] + b
    cglob = j * bc + jax.lax.broadcasted_iota(jnp.int32, (bc, 1), 0)
    l = jnp.where(cglob < num_classes, l, -1e30)

    K = jnp.abs(w) + jnp.abs(b)
    s_tile = jnp.sum(jnp.exp(l - K), axis=0)
    m_tile = jnp.max(l, axis=0)
    idx_tile = jnp.min(jnp.where(l == m_tile[None, :], cglob, jnp.int32(2**30)),
                       axis=0)
    clsv = cls_ref[...]
    t_tile = jnp.sum(jnp.where(cglob == clsv[None, :], l, 0.0), axis=0)

    @pl.when(j == 0)
    def _():
        m_ref[...] = m_tile
        a_ref[...] = idx_tile
        s_ref[...] = s_tile
        t_ref[...] = t_tile

    @pl.when(j > 0)
    def _():
        m_old = m_ref[...]
        upd = m_tile > m_old
        m_ref[...] = jnp.maximum(m_old, m_tile)
        a_ref[...] = jnp.where(upd, idx_tile, a_ref[...])
        s_ref[...] = s_ref[...] + s_tile
        t_ref[...] = t_ref[...] + t_tile

    @pl.when(j == nc - 1)
    def _():
        row_loss = jnp.log(s_ref[...]) + K - t_ref[...]
        part_loss = jnp.sum(row_loss)
        part_acc = jnp.sum((a_ref[...] == clsv).astype(jnp.float32))

        @pl.when(i == 0)
        def _():
            loss_ref[0, 0] = part_loss
            acc_ref[0, 0] = part_acc

        @pl.when(i > 0)
        def _():
            loss_ref[0, 0] = loss_ref[0, 0] + part_loss
            acc_ref[0, 0] = acc_ref[0, 0] + part_acc


def _fused_loss(cls, wb, rp, rq, phit, plot, qhi, qlo, num_classes,
                interpret=False):
    dim, batch = phit.shape
    c_pad = qhi.shape[0]
    bb = _pick_bb(batch)
    bc = _pick_bc(c_pad)
    nb = batch // bb
    nc = c_pad // bc
    body = functools.partial(_main_body, nc, num_classes)
    loss_sum, acc_sum = pl.pallas_call(
        body,
        grid=(nb, nc),
        in_specs=[
            pl.BlockSpec((bb,), lambda i, j: (i,)),
            pl.BlockSpec(memory_space=pltpu.SMEM),
            pl.BlockSpec((bb,), lambda i, j: (i,)),
            pl.BlockSpec((1, bc), lambda i, j: (0, j)),
            pl.BlockSpec((dim, bb), lambda i, j: (0, i)),
            pl.BlockSpec((dim, bb), lambda i, j: (0, i)),
            pl.BlockSpec((bc, dim), lambda i, j: (j, 0)),
            pl.BlockSpec((bc, dim), lambda i, j: (j, 0)),
        ],
        out_specs=[
            pl.BlockSpec(memory_space=pltpu.SMEM),
            pl.BlockSpec(memory_space=pltpu.SMEM),
        ],
        out_shape=[
            jax.ShapeDtypeStruct((1, 1), jnp.float32),
            jax.ShapeDtypeStruct((1, 1), jnp.float32),
        ],
        scratch_shapes=[
            pltpu.VMEM((bb,), jnp.float32),
            pltpu.VMEM((bb,), jnp.int32),
            pltpu.VMEM((bb,), jnp.float32),
            pltpu.VMEM((bb,), jnp.float32),
        ],
        compiler_params=pltpu.CompilerParams(
            dimension_semantics=("arbitrary", "arbitrary"),
        ),
        interpret=interpret,
    )(cls, wb, rp, rq.reshape(1, c_pad), phit, plot, qhi, qlo)
    nloss = loss_sum[0, 0] / batch
    prec1 = acc_sum[0, 0] / batch * 100.0
    return nloss, prec1


def kernel(x, epoch, classes, w, b, queue, queue_ptr):
    batch = x.shape[0]
    num_classes = queue.shape[0]
    cls = classes[0]
    iota = jnp.arange(batch, dtype=jnp.int32)
    winner = jnp.full((num_classes,), -1, jnp.int32).at[cls].max(iota)
    covered = winner >= 0
    anchors = x[:, 1, :]
    q_eff = jnp.where(covered[:, None], anchors[jnp.maximum(winner, 0)],
                      queue[:, 0, :])
    c_pad = ((num_classes + 383) // 384) * 384
    q_eff = jnp.pad(q_eff, ((0, c_pad - num_classes), (0, 0)))

    pos_t = x[:, 0, :].T
    rp = 1.0 / jnp.maximum(jnp.linalg.norm(pos_t, axis=0), 1e-8)
    rq = 1.0 / jnp.maximum(jnp.linalg.norm(q_eff, axis=1), 1e-8)
    phit = pos_t.astype(jnp.bfloat16)
    plot = (pos_t - phit.astype(jnp.float32)).astype(jnp.bfloat16)
    qhi = q_eff.astype(jnp.bfloat16)
    qlo = (q_eff - qhi.astype(jnp.float32)).astype(jnp.bfloat16)

    wb = jnp.stack([w.astype(jnp.float32), b.astype(jnp.float32)])
    return _fused_loss(cls, wb, rp, rq, phit, plot, qhi, qlo, num_classes)


# speedup vs baseline: 1.2249x; 1.1000x over previous
import functools

import jax
import jax.numpy as jnp
from jax import lax
from jax.experimental import pallas as pl
from jax.experimental.pallas import tpu as pltpu
from jax.experimental.pallas import tpu_sc as plsc

DIM = 4096
BATCH = 4096
NUM_CLASSES = DIM + 1
C_PAD = 4608
NW = 32
RPW = 136
C_OWNED = NW * RPW
CH = 8
NCHUNK = RPW // CH

BB = 1024
BC = 384


def _prologue_body(x_ref, phit_ref, plot_ref, rp_ref):
    P = x_ref[:, 0, 0, :]
    ssq = jnp.sum(P * P, axis=1)
    rp_ref[...] = 1.0 / jnp.maximum(jnp.sqrt(ssq), 1e-8)
    Pt = P.T
    hi = Pt.astype(jnp.bfloat16)
    phit_ref[...] = hi
    plot_ref[...] = (Pt - hi.astype(jnp.float32)).astype(jnp.bfloat16)


def _prologue(x4):
    batch, _, _, dim = x4.shape
    bb2 = 512
    nb = batch // bb2
    return pl.pallas_call(
        _prologue_body,
        grid=(nb,),
        in_specs=[pl.BlockSpec((bb2, 1, 1, dim), lambda i: (i, 0, 0, 0))],
        out_specs=[
            pl.BlockSpec((dim, bb2), lambda i: (0, i)),
            pl.BlockSpec((dim, bb2), lambda i: (0, i)),
            pl.BlockSpec((bb2,), lambda i: (i,)),
        ],
        out_shape=[
            jax.ShapeDtypeStruct((dim, batch), jnp.bfloat16),
            jax.ShapeDtypeStruct((dim, batch), jnp.bfloat16),
            jax.ShapeDtypeStruct((batch,), jnp.float32),
        ],
        compiler_params=pltpu.CompilerParams(
            dimension_semantics=("arbitrary",),
        ),
    )(x4)


def _sc_body(x2, queue2, srcg, dstg, out, buf, srcv, dstv, sem):
    nc_mesh = 2
    wid = lax.axis_index("s") * nc_mesh + lax.axis_index("c")
    c0 = wid * RPW

    for ch in range(NCHUNK):
        start = c0 + ch * CH

        @pl.when(start + CH <= NUM_CLASSES)
        def _():
            pltpu.sync_copy(queue2.at[pl.ds(start, CH)], buf)
            pltpu.sync_copy(buf, out.at[pl.ds(start, CH)])

        @pl.when(start == NUM_CLASSES - 1)
        def _():
            pltpu.sync_copy(queue2.at[pl.ds(start, 1)],
                            buf.at[pl.ds(0, 1)])
            pltpu.sync_copy(buf.at[pl.ds(0, 1)], out.at[pl.ds(start, 1)])

    pltpu.sync_copy(srcg.at[wid], srcv)
    pltpu.sync_copy(dstg.at[wid], dstv)
    for ch in range(NCHUNK):
        pltpu.async_copy(x2.at[srcv.at[ch]], buf, sem).wait()
        pltpu.async_copy(buf, out.at[dstv.at[ch]], sem).wait()


def _sc_build_queue(x2, queue2, srcg, dstg):
    dim = x2.shape[1]
    mesh = plsc.VectorSubcoreMesh(core_axis_name="c", subcore_axis_name="s")
    k = pl.kernel(
        _sc_body,
        out_type=jax.ShapeDtypeStruct((C_PAD, dim), jnp.float32),
        mesh=mesh,
        scratch_types=[
            pltpu.VMEM((CH, dim), jnp.float32),
            pltpu.VMEM((NCHUNK, CH), jnp.int32),
            pltpu.VMEM((NCHUNK, CH), jnp.int32),
            pltpu.SemaphoreType.DMA,
        ],
    )
    return k(x2, queue2, srcg, dstg)


def _main_body(nc, num_classes, cls_ref, wb_ref, rp_ref,
               phit_ref, plot_ref, q_ref, loss_ref, acc_ref,
               m_ref, a_ref, s_ref, t_ref):
    i = pl.program_id(0)
    j = pl.program_id(1)
    w = wb_ref[0]
    b = wb_ref[1]
    bc = q_ref.shape[0]

    qf = q_ref[...]
    qhi = qf.astype(jnp.bfloat16)
    qlo = (qf - qhi.astype(jnp.float32)).astype(jnp.bfloat16)
    ssq = jnp.sum(qf * qf, axis=1)
    rq = 1.0 / jnp.maximum(jnp.sqrt(ssq), 1e-8)

    nn = (((1,), (0,)), ((), ()))
    S = jax.lax.dot_general(qhi, phit_ref[...], nn,
                            preferred_element_type=jnp.float32)
    S += jax.lax.dot_general(qlo, phit_ref[...], nn,
                             preferred_element_type=jnp.float32)
    S += jax.lax.dot_general(qhi, plot_ref[...], nn,
                             preferred_element_type=jnp.float32)

    rp = rp_ref[...]
    l = (S * rq[:, None]) * (rp * w)[None, :] + b
    cglob = j * bc + jax.lax.broadcasted_iota(jnp.int32, (bc, 1), 0)
    l = jnp.where(cglob < num_classes, l, -1e30)

    K = jnp.abs(w) + jnp.abs(b)
    s_tile = jnp.sum(jnp.exp(l - K), axis=0)
    m_tile = jnp.max(l, axis=0)
    idx_tile = jnp.min(jnp.where(l == m_tile[None, :], cglob, jnp.int32(2**30)),
                       axis=0)
    clsv = cls_ref[...]
    t_tile = jnp.sum(jnp.where(cglob == clsv[None, :], l, 0.0), axis=0)

    @pl.when(j == 0)
    def _():
        m_ref[...] = m_tile
        a_ref[...] = idx_tile
        s_ref[...] = s_tile
        t_ref[...] = t_tile

    @pl.when(j > 0)
    def _():
        m_old = m_ref[...]
        upd = m_tile > m_old
        m_ref[...] = jnp.maximum(m_old, m_tile)
        a_ref[...] = jnp.where(upd, idx_tile, a_ref[...])
        s_ref[...] = s_ref[...] + s_tile
        t_ref[...] = t_ref[...] + t_tile

    @pl.when(j == nc - 1)
    def _():
        row_loss = jnp.log(s_ref[...]) + K - t_ref[...]
        part_loss = jnp.sum(row_loss)
        part_acc = jnp.sum((a_ref[...] == clsv).astype(jnp.float32))

        @pl.when(i == 0)
        def _():
            loss_ref[0, 0] = part_loss
            acc_ref[0, 0] = part_acc

        @pl.when(i > 0)
        def _():
            loss_ref[0, 0] = loss_ref[0, 0] + part_loss
            acc_ref[0, 0] = acc_ref[0, 0] + part_acc


def _fused_loss(cls, wb, rp, phit, plot, q_eff, num_classes,
                interpret=False):
    dim, batch = phit.shape
    c_pad = q_eff.shape[0]
    bb = BB if batch % BB == 0 else batch
    bc = BC if c_pad % BC == 0 else 128
    nb = batch // bb
    nc = c_pad // bc
    body = functools.partial(_main_body, nc, num_classes)
    loss_sum, acc_sum = pl.pallas_call(
        body,
        grid=(nb, nc),
        in_specs=[
            pl.BlockSpec((bb,), lambda i, j: (i,)),
            pl.BlockSpec(memory_space=pltpu.SMEM),
            pl.BlockSpec((bb,), lambda i, j: (i,)),
            pl.BlockSpec((dim, bb), lambda i, j: (0, i)),
            pl.BlockSpec((dim, bb), lambda i, j: (0, i)),
            pl.BlockSpec((bc, dim), lambda i, j: (j, 0)),
        ],
        out_specs=[
            pl.BlockSpec(memory_space=pltpu.SMEM),
            pl.BlockSpec(memory_space=pltpu.SMEM),
        ],
        out_shape=[
            jax.ShapeDtypeStruct((1, 1), jnp.float32),
            jax.ShapeDtypeStruct((1, 1), jnp.float32),
        ],
        scratch_shapes=[
            pltpu.VMEM((bb,), jnp.float32),
            pltpu.VMEM((bb,), jnp.int32),
            pltpu.VMEM((bb,), jnp.float32),
            pltpu.VMEM((bb,), jnp.float32),
        ],
        compiler_params=pltpu.CompilerParams(
            dimension_semantics=("arbitrary", "arbitrary"),
        ),
        interpret=interpret,
    )(cls, wb, rp, phit, plot, q_eff)
    nloss = loss_sum[0, 0] / batch
    prec1 = acc_sum[0, 0] / batch * 100.0
    return nloss, prec1


def kernel(x, epoch, classes, w, b, queue, queue_ptr):
    batch = x.shape[0]
    dim = x.shape[2]
    num_classes = queue.shape[0]
    cls = classes[0]

    iota = jnp.arange(batch, dtype=jnp.int32)
    winner = jnp.full((num_classes,), -1, jnp.int32).at[cls].max(iota)
    covered = winner >= 0
    cov = jnp.zeros((C_OWNED,), jnp.bool_).at[:num_classes].set(covered)
    cw = cov.reshape(NW, RPW)
    posl = jnp.cumsum(cw.astype(jnp.int32), axis=1) - cw.astype(jnp.int32)
    colidx = jnp.where(cw, posl, RPW)
    widx = jnp.arange(NW, dtype=jnp.int32)[:, None]
    cls_ids = jnp.arange(C_OWNED, dtype=jnp.int32).reshape(NW, RPW)
    trash = (C_OWNED + jnp.arange(NW, dtype=jnp.int32))[:, None]
    dstg = jnp.broadcast_to(trash, (NW, RPW + 1)).at[widx, colidx].set(
        cls_ids, mode="drop")[:, :RPW]
    srcvals = jnp.zeros((C_OWNED,), jnp.int32).at[:num_classes].set(
        2 * jnp.maximum(winner, 0) + 1)
    srcg = jnp.ones((NW, RPW + 1), jnp.int32).at[widx, colidx].set(
        srcvals.reshape(NW, RPW), mode="drop")[:, :RPW]
    srcg = srcg.reshape(NW, NCHUNK, CH)
    dstg = dstg.reshape(NW, NCHUNK, CH)

    x2 = x.reshape(2 * batch, dim)
    queue2 = queue.reshape(num_classes, dim)

    q_eff = _sc_build_queue(x2, queue2, srcg, dstg)

    x4 = x.reshape(batch, 2, 1, dim)
    phit, plot, rp = _prologue(x4)

    wb = jnp.stack([w.astype(jnp.float32), b.astype(jnp.float32)])
    return _fused_loss(cls, wb, rp, phit, plot, q_eff, num_classes)
